# Initial kernel scaffold; baseline (speedup 1.0000x reference)
import jax
import jax.numpy as jnp
from jax.experimental import pallas as pl


def kernel(inputs, state, weights_0, bias_0, weights_1, bias_1, lap_vals, edge_index):
    raise NotImplementedError("write your pallas kernel here")


# speedup vs baseline: 1.0303x; 1.0303x over previous
import jax
import jax.numpy as jnp
from jax.experimental import pallas as pl

N = 10000
U = 32
B = 8


def _gate1(s_ref, st_ref, rs_ref, u_ref):
    v = jax.nn.sigmoid(s_ref[...])
    r = v[:, :U]
    u = v[:, U:]
    rs_ref[...] = r * st_ref[...]
    u_ref[...] = u


def _gate2(c_ref, st_ref, u_ref, ns_ref):
    c = jnp.tanh(c_ref[...])
    u = u_ref[...]
    ns_ref[...] = u * st_ref[...] + (1.0 - u) * c


def kernel(inputs, state, weights_0, bias_0, weights_1, bias_1, lap_vals, edge_index):
    src = edge_index[0].astype(jnp.int32)
    dst = edge_index[1].astype(jnp.int32)

    def gc_raw(inp, st, W, b):
        x = jnp.concatenate([inp.reshape(B, N, 1), st.reshape(B, N, U)], axis=2)
        in_size = x.shape[2]
        x0 = jnp.transpose(x, (1, 2, 0)).reshape(N, in_size * B)
        agg = jax.ops.segment_sum(lap_vals[:, None] * x0[src], dst, num_segments=N)
        x1 = x0 - agg
        x1 = jnp.transpose(x1.reshape(N, in_size, B), (2, 0, 1)).reshape(-1, in_size)
        return x1 @ W + b

    ROWS = B * N
    BLK = 8000
    grid = (ROWS // BLK,)

    s0 = gc_raw(inputs, state, weights_0, bias_0).reshape(ROWS, 2 * U)
    st2 = state.reshape(ROWS, U)
    rs, u = pl.pallas_call(
        _gate1,
        grid=grid,
        in_specs=[
            pl.BlockSpec((BLK, 2 * U), lambda i: (i, 0)),
            pl.BlockSpec((BLK, U), lambda i: (i, 0)),
        ],
        out_specs=[
            pl.BlockSpec((BLK, U), lambda i: (i, 0)),
            pl.BlockSpec((BLK, U), lambda i: (i, 0)),
        ],
        out_shape=[
            jax.ShapeDtypeStruct((ROWS, U), jnp.float32),
            jax.ShapeDtypeStruct((ROWS, U), jnp.float32),
        ],
    )(s0, st2)

    c_raw = gc_raw(inputs, rs.reshape(B, N * U), weights_1, bias_1).reshape(ROWS, U)
    new_state = pl.pallas_call(
        _gate2,
        grid=grid,
        in_specs=[
            pl.BlockSpec((BLK, U), lambda i: (i, 0)),
            pl.BlockSpec((BLK, U), lambda i: (i, 0)),
            pl.BlockSpec((BLK, U), lambda i: (i, 0)),
        ],
        out_specs=pl.BlockSpec((BLK, U), lambda i: (i, 0)),
        out_shape=jax.ShapeDtypeStruct((ROWS, U), jnp.float32),
    )(c_raw, st2, u)

    return new_state.reshape(B, N * U)


# speedup vs baseline: 2.3809x; 2.3110x over previous
import functools

import jax
import jax.numpy as jnp
from jax import lax
from jax.experimental import pallas as pl
from jax.experimental.pallas import tpu as pltpu
import jax.experimental.pallas.tpu_sc as plsc

N = 10000
U = 32
B = 8
E = 160000

NC = 2
NS = 16
L = 16

CHK = 128
CPW = 80
NROW = NS * CPW
EPAD = NROW * CHK
NP = 10240
RPT = NP // NS
CPW2 = CPW // 2

RBLK = 1000
GRID = N // RBLK


def _sc_agg(srccat, dstr, lapr, table, z128,
            agg,
            accum, idxs, dstb, lapb, rowb, sem1):
    c = lax.axis_index("c")
    s = lax.axis_index("s")
    pltpu.sync_copy(z128.at[pl.ds(s * RPT, RPT)], accum.at[pl.ds(s * RPT, RPT)])
    pltpu.sync_copy(srccat.at[pl.ds(c * NROW + s * CPW, CPW)], idxs)
    pltpu.sync_copy(dstr.at[pl.ds(s * CPW, CPW)], dstb)
    pltpu.sync_copy(lapr.at[pl.ds(s * CPW, CPW)], lapb)
    plsc.subcore_barrier()

    def chunk(j, carry):
        pltpu.async_copy(table.at[idxs.at[j]], rowb, sem1).wait()
        def body(g, cin):
            lv16 = lapb[j, pl.ds(g * L, L)]
            for e16 in range(L):
                lvb = jnp.broadcast_to(lax.slice(lv16, (e16,), (e16 + 1,)), (L,))
                e = g * L + e16
                for k in range(8):
                    sl = (e, pl.ds(k * L, L))
                    rowb[sl] = rowb[sl] * lvb
            return cin
        lax.fori_loop(0, CHK // L, body, 0)
        pltpu.sync_copy(rowb, accum.at[dstb.at[j]], add=True)
        return carry

    lax.fori_loop(0, CPW, chunk, 0)
    plsc.subcore_barrier()
    pltpu.sync_copy(accum.at[pl.ds(s * RPT, RPT)],
                    agg.at[c, pl.ds(s * RPT, RPT)])


def _sc_xi(srcr, dstr, lapr, xip128, z128,
           aggi,
           accumi, idxi, dsti, lapi, rowxi, sem1):
    c = lax.axis_index("c")
    s = lax.axis_index("s")
    pltpu.sync_copy(z128.at[pl.ds(s * RPT, RPT)], accumi.at[pl.ds(s * RPT, RPT)])
    xoff = c * (NROW // 2) + s * CPW2
    pltpu.sync_copy(srcr.at[pl.ds(xoff, CPW2)], idxi)
    pltpu.sync_copy(dstr.at[pl.ds(xoff, CPW2)], dsti)
    pltpu.sync_copy(lapr.at[pl.ds(xoff, CPW2)], lapi)
    plsc.subcore_barrier()

    def chunk(j, carry):
        pltpu.async_copy(xip128.at[idxi.at[j]], rowxi, sem1).wait()
        def body(g, cin):
            lv16 = lapi[j, pl.ds(g * L, L)]
            for e16 in range(L):
                lvb = jnp.broadcast_to(lax.slice(lv16, (e16,), (e16 + 1,)), (L,))
                e = g * L + e16
                rowxi[e, pl.ds(0, L)] = rowxi[e, pl.ds(0, L)] * lvb
            return cin
        lax.fori_loop(0, CHK // L, body, 0)
        pltpu.sync_copy(rowxi, accumi.at[dsti.at[j]], add=True)
        return carry

    lax.fori_loop(0, CPW2, chunk, 0)
    plsc.subcore_barrier()
    pltpu.sync_copy(accumi.at[pl.ds(s * RPT, RPT)],
                    aggi.at[c, pl.ds(s * RPT, RPT)])


def _make_agg():
    mesh = plsc.VectorSubcoreMesh(core_axis_name="c", subcore_axis_name="s",
                                  num_cores=NC, num_subcores=NS)
    return pl.kernel(
        _sc_agg,
        out_type=jax.ShapeDtypeStruct((NC, NP, 128), jnp.float32),
        mesh=mesh,
        scratch_types=[
            pltpu.VMEM_SHARED((NP, 128), jnp.float32),
            pltpu.VMEM((CPW, CHK), jnp.int32),
            pltpu.VMEM((CPW, CHK), jnp.int32),
            pltpu.VMEM((CPW, CHK), jnp.float32),
            pltpu.VMEM((CHK, 128), jnp.float32),
            pltpu.SemaphoreType.DMA,
        ],
    )


def _make_xi():
    mesh = plsc.VectorSubcoreMesh(core_axis_name="c", subcore_axis_name="s",
                                  num_cores=NC, num_subcores=NS)
    return pl.kernel(
        _sc_xi,
        out_type=jax.ShapeDtypeStruct((NC, NP, 128), jnp.float32),
        mesh=mesh,
        scratch_types=[
            pltpu.VMEM_SHARED((NP, 128), jnp.float32),
            pltpu.VMEM((CPW2, CHK), jnp.int32),
            pltpu.VMEM((CPW2, CHK), jnp.int32),
            pltpu.VMEM((CPW2, CHK), jnp.float32),
            pltpu.VMEM((CHK, 128), jnp.float32),
            pltpu.SemaphoreType.DMA,
        ],
    )


def _prep_body(st_ref, xst_ref):
    for b in range(B):
        xst_ref[:, b * U:(b + 1) * U] = st_ref[b]


def _tc1_body(xip_ref, aggi0_ref, aggi1_ref, st_ref, agg0_ref, agg1_ref,
              w0s_ref, w0i_ref, b0_ref, w1s_ref, w1i_ref,
              z_ref, u_ref):
    xin = (xip_ref[:, 0:B] - aggi0_ref[:, 0:B] - aggi1_ref[:, 0:B])
    w0s = w0s_ref[...]
    w0i = w0i_ref[...]
    b0 = b0_ref[...]
    w1s = w1s_ref[...]
    w1i = w1i_ref[...]
    for b in range(B):
        half = agg0_ref if b < 4 else agg1_ref
        agg_b = half[:, (b % 4) * U:(b % 4 + 1) * U]
        xs_b = st_ref[b]
        x1s = xs_b - agg_b
        xin_b = xin[:, b:b + 1]
        y = (jnp.dot(x1s, w0s, preferred_element_type=jnp.float32)
             + xin_b * w0i + b0)
        v = jax.nn.sigmoid(y)
        r_b = v[:, :U]
        u_b = v[:, U:]
        h_b = r_b * xs_b
        z_b = (jnp.dot(h_b, w1s, preferred_element_type=jnp.float32)
               + xip_ref[:, b:b + 1] * w1i)
        z_ref[:, b * U:(b + 1) * U] = z_b
        u_ref[:, b * U:(b + 1) * U] = u_b


def _tc2_body(z_ref, az0_ref, az1_ref, u_ref, st_ref, b1_ref, ns_ref):
    b1 = b1_ref[...]
    for b in range(B):
        half = az0_ref if b < 4 else az1_ref
        aggz_b = half[:, (b % 4) * U:(b % 4 + 1) * U]
        z_b = z_ref[:, b * U:(b + 1) * U]
        c_b = jnp.tanh(z_b - aggz_b + b1)
        u_b = u_ref[:, b * U:(b + 1) * U]
        ns_ref[b] = u_b * st_ref[b] + (1.0 - u_b) * c_b


def kernel(inputs, state, weights_0, bias_0, weights_1, bias_1, lap_vals,
           edge_index):
    f32 = jnp.float32
    src = edge_index[0].astype(jnp.int32)
    dst = edge_index[1].astype(jnp.int32)
    lap = lap_vals.astype(f32)

    pad = EPAD - E
    zi = jnp.zeros((pad,), jnp.int32)
    src_p = jnp.concatenate([src, zi])
    dst_p = jnp.concatenate([dst, zi])
    lap_p = jnp.concatenate([lap, jnp.zeros((pad,), f32)])
    srccat = jnp.concatenate([2 * src_p, 2 * src_p + 1]).reshape(2 * NROW, CHK)
    srcr = src_p.reshape(NROW, CHK)
    dstr = dst_p.reshape(NROW, CHK)
    lapr = lap_p.reshape(NROW, CHK)

    xip16 = jnp.concatenate(
        [inputs.reshape(B, N).T, jnp.zeros((N, 16 - B), f32)], axis=1)
    xip128 = jnp.concatenate(
        [inputs.reshape(B, N).T, jnp.zeros((N, 128 - B), f32)], axis=1)
    state3 = state.reshape(B, N, U)
    z128 = jnp.zeros((NP, 128), f32)

    w0s = weights_0[1:]
    w0i = weights_0[0:1]
    b0r = bias_0.reshape(1, 2 * U)
    w1s = weights_1[1:]
    w1i = weights_1[0:1]
    b1r = bias_1.reshape(1, U)

    xst = pl.pallas_call(
        _prep_body,
        grid=(GRID,),
        in_specs=[pl.BlockSpec((B, RBLK, U), lambda i: (0, i, 0))],
        out_specs=pl.BlockSpec((RBLK, B * U), lambda i: (i, 0)),
        out_shape=jax.ShapeDtypeStruct((N, B * U), f32),
    )(state3)
    xst2 = xst.reshape(2 * N, 128)

    aggs = _make_agg()(srccat, dstr, lapr, xst2, z128)
    aggi = _make_xi()(srcr, dstr, lapr, xip128, z128)

    z1, uu = pl.pallas_call(
        _tc1_body,
        grid=(GRID,),
        in_specs=[
            pl.BlockSpec((RBLK, 16), lambda i: (i, 0)),
            pl.BlockSpec((RBLK, 128), lambda i: (i, 0)),
            pl.BlockSpec((RBLK, 128), lambda i: (i, 0)),
            pl.BlockSpec((B, RBLK, U), lambda i: (0, i, 0)),
            pl.BlockSpec((RBLK, 128), lambda i: (i, 0)),
            pl.BlockSpec((RBLK, 128), lambda i: (i, 0)),
            pl.BlockSpec((U, 2 * U), lambda i: (0, 0)),
            pl.BlockSpec((1, 2 * U), lambda i: (0, 0)),
            pl.BlockSpec((1, 2 * U), lambda i: (0, 0)),
            pl.BlockSpec((U, U), lambda i: (0, 0)),
            pl.BlockSpec((1, U), lambda i: (0, 0)),
        ],
        out_specs=[
            pl.BlockSpec((RBLK, B * U), lambda i: (i, 0)),
            pl.BlockSpec((RBLK, B * U), lambda i: (i, 0)),
        ],
        out_shape=[
            jax.ShapeDtypeStruct((N, B * U), f32),
            jax.ShapeDtypeStruct((N, B * U), f32),
        ],
    )(xip16, aggi[0], aggi[1], state3, aggs[0], aggs[1], w0s, w0i, b0r, w1s, w1i)

    z2 = z1.reshape(2 * N, 128)
    aggz = _make_agg()(srccat, dstr, lapr, z2, z128)

    ns = pl.pallas_call(
        _tc2_body,
        grid=(GRID,),
        in_specs=[
            pl.BlockSpec((RBLK, B * U), lambda i: (i, 0)),
            pl.BlockSpec((RBLK, 128), lambda i: (i, 0)),
            pl.BlockSpec((RBLK, 128), lambda i: (i, 0)),
            pl.BlockSpec((RBLK, B * U), lambda i: (i, 0)),
            pl.BlockSpec((B, RBLK, U), lambda i: (0, i, 0)),
            pl.BlockSpec((1, U), lambda i: (0, 0)),
        ],
        out_specs=pl.BlockSpec((B, RBLK, U), lambda i: (0, i, 0)),
        out_shape=jax.ShapeDtypeStruct((B, N, U), f32),
    )(z1, aggz[0], aggz[1], uu, state3, b1r)

    return ns.reshape(B, N * U)


# speedup vs baseline: 2.4960x; 1.0483x over previous
import functools

import jax
import jax.numpy as jnp
from jax import lax
from jax.experimental import pallas as pl
from jax.experimental.pallas import tpu as pltpu
import jax.experimental.pallas.tpu_sc as plsc

N = 10000
U = 32
B = 8
E = 160000

NC = 2
NS = 16
L = 16

CHK = 128
CPW = 80
NROW = NS * CPW
EPAD = NROW * CHK
NP = 10240
RPT = NP // NS
CPW2 = CPW // 2

RBLK = 1000
GRID = N // RBLK


BLK_CH = 16
NBLK = CPW // BLK_CH
BLK_CH2 = 8
NBLK2 = CPW2 // BLK_CH2


def _scale_chunk(rowb, lapb, j, nvec=8):
    def body(g, cin):
        lv16 = lapb[j, pl.ds(g * L, L)]
        for e16 in range(L):
            lvb = jnp.broadcast_to(lax.slice(lv16, (e16,), (e16 + 1,)), (L,))
            e = g * L + e16
            for k in range(nvec):
                sl = (e, pl.ds(k * L, L))
                rowb[sl] = rowb[sl] * lvb
        return cin
    lax.fori_loop(0, CHK // L, body, 0)


def _sc_agg(srccat, dstr, lapr, table, z128,
            agg,
            accum, idxs, dstb, lapb, rowb0, rowb1, gs0, gs1, ss0, ss1):
    c = lax.axis_index("c")
    s = lax.axis_index("s")
    pltpu.sync_copy(z128.at[pl.ds(s * RPT, RPT)], accum.at[pl.ds(s * RPT, RPT)])
    plsc.subcore_barrier()

    def block(bk, carry):
        pltpu.sync_copy(
            srccat.at[pl.ds(c * NROW + s * CPW + bk * BLK_CH, BLK_CH)], idxs)
        pltpu.sync_copy(dstr.at[pl.ds(s * CPW + bk * BLK_CH, BLK_CH)], dstb)
        pltpu.sync_copy(lapr.at[pl.ds(s * CPW + bk * BLK_CH, BLK_CH)], lapb)

        def pair(p, cin):
            j0 = 2 * p
            j1 = 2 * p + 1
            g0 = pltpu.async_copy(table.at[idxs.at[j0]], rowb0, gs0)
            g1 = pltpu.async_copy(table.at[idxs.at[j1]], rowb1, gs1)
            g0.wait()
            _scale_chunk(rowb0, lapb, j0)
            s0 = pltpu.async_copy(rowb0, accum.at[dstb.at[j0]], ss0, add=True)
            g1.wait()
            _scale_chunk(rowb1, lapb, j1)
            s1 = pltpu.async_copy(rowb1, accum.at[dstb.at[j1]], ss1, add=True)
            s0.wait()
            s1.wait()
            return cin

        lax.fori_loop(0, BLK_CH // 2, pair, 0)
        return carry

    lax.fori_loop(0, NBLK, block, 0)
    plsc.subcore_barrier()
    pltpu.sync_copy(accum.at[pl.ds(s * RPT, RPT)],
                    agg.at[c, pl.ds(s * RPT, RPT)])


def _sc_xi(srcr, dstr, lapr, xip128, z128,
           aggi,
           accumi, idxi, dsti, lapi, rowx0, rowx1, gs0, gs1, ss0, ss1):
    c = lax.axis_index("c")
    s = lax.axis_index("s")
    pltpu.sync_copy(z128.at[pl.ds(s * RPT, RPT)], accumi.at[pl.ds(s * RPT, RPT)])
    plsc.subcore_barrier()

    def block(bk, carry):
        xoff = c * (NROW // 2) + s * CPW2 + bk * BLK_CH2
        pltpu.sync_copy(srcr.at[pl.ds(xoff, BLK_CH2)], idxi)
        pltpu.sync_copy(dstr.at[pl.ds(xoff, BLK_CH2)], dsti)
        pltpu.sync_copy(lapr.at[pl.ds(xoff, BLK_CH2)], lapi)

        def pair(p, cin):
            j0 = 2 * p
            j1 = 2 * p + 1
            g0 = pltpu.async_copy(xip128.at[idxi.at[j0]], rowx0, gs0)
            g1 = pltpu.async_copy(xip128.at[idxi.at[j1]], rowx1, gs1)
            g0.wait()
            _scale_chunk(rowx0, lapi, j0, nvec=1)
            s0 = pltpu.async_copy(rowx0, accumi.at[dsti.at[j0]], ss0, add=True)
            g1.wait()
            _scale_chunk(rowx1, lapi, j1, nvec=1)
            s1 = pltpu.async_copy(rowx1, accumi.at[dsti.at[j1]], ss1, add=True)
            s0.wait()
            s1.wait()
            return cin

        lax.fori_loop(0, BLK_CH2 // 2, pair, 0)
        return carry

    lax.fori_loop(0, NBLK2, block, 0)
    plsc.subcore_barrier()
    pltpu.sync_copy(accumi.at[pl.ds(s * RPT, RPT)],
                    aggi.at[c, pl.ds(s * RPT, RPT)])


def _make_agg():
    mesh = plsc.VectorSubcoreMesh(core_axis_name="c", subcore_axis_name="s",
                                  num_cores=NC, num_subcores=NS)
    return pl.kernel(
        _sc_agg,
        out_type=jax.ShapeDtypeStruct((NC, NP, 128), jnp.float32),
        mesh=mesh,
        scratch_types=[
            pltpu.VMEM_SHARED((NP, 128), jnp.float32),
            pltpu.VMEM((BLK_CH, CHK), jnp.int32),
            pltpu.VMEM((BLK_CH, CHK), jnp.int32),
            pltpu.VMEM((BLK_CH, CHK), jnp.float32),
            pltpu.VMEM((CHK, 128), jnp.float32),
            pltpu.VMEM((CHK, 128), jnp.float32),
            pltpu.SemaphoreType.DMA,
            pltpu.SemaphoreType.DMA,
            pltpu.SemaphoreType.DMA,
            pltpu.SemaphoreType.DMA,
        ],
    )


def _make_xi():
    mesh = plsc.VectorSubcoreMesh(core_axis_name="c", subcore_axis_name="s",
                                  num_cores=NC, num_subcores=NS)
    return pl.kernel(
        _sc_xi,
        out_type=jax.ShapeDtypeStruct((NC, NP, 128), jnp.float32),
        mesh=mesh,
        scratch_types=[
            pltpu.VMEM_SHARED((NP, 128), jnp.float32),
            pltpu.VMEM((BLK_CH2, CHK), jnp.int32),
            pltpu.VMEM((BLK_CH2, CHK), jnp.int32),
            pltpu.VMEM((BLK_CH2, CHK), jnp.float32),
            pltpu.VMEM((CHK, 128), jnp.float32),
            pltpu.VMEM((CHK, 128), jnp.float32),
            pltpu.SemaphoreType.DMA,
            pltpu.SemaphoreType.DMA,
            pltpu.SemaphoreType.DMA,
            pltpu.SemaphoreType.DMA,
        ],
    )


def _prep_body(st_ref, xst_ref):
    for b in range(B):
        xst_ref[:, b * U:(b + 1) * U] = st_ref[b]


def _tc1_body(xip_ref, aggi0_ref, aggi1_ref, st_ref, agg0_ref, agg1_ref,
              w0s_ref, w0i_ref, b0_ref, w1s_ref, w1i_ref,
              z_ref, u_ref):
    xin = (xip_ref[:, 0:B] - aggi0_ref[:, 0:B] - aggi1_ref[:, 0:B])
    w0s = w0s_ref[...]
    w0i = w0i_ref[...]
    b0 = b0_ref[...]
    w1s = w1s_ref[...]
    w1i = w1i_ref[...]
    for b in range(B):
        half = agg0_ref if b < 4 else agg1_ref
        agg_b = half[:, (b % 4) * U:(b % 4 + 1) * U]
        xs_b = st_ref[b]
        x1s = xs_b - agg_b
        xin_b = xin[:, b:b + 1]
        y = (jnp.dot(x1s, w0s, preferred_element_type=jnp.float32)
             + xin_b * w0i + b0)
        v = jax.nn.sigmoid(y)
        r_b = v[:, :U]
        u_b = v[:, U:]
        h_b = r_b * xs_b
        z_b = (jnp.dot(h_b, w1s, preferred_element_type=jnp.float32)
               + xip_ref[:, b:b + 1] * w1i)
        z_ref[:, b * U:(b + 1) * U] = z_b
        u_ref[:, b * U:(b + 1) * U] = u_b


def _tc2_body(z_ref, az0_ref, az1_ref, u_ref, st_ref, b1_ref, ns_ref):
    b1 = b1_ref[...]
    for b in range(B):
        half = az0_ref if b < 4 else az1_ref
        aggz_b = half[:, (b % 4) * U:(b % 4 + 1) * U]
        z_b = z_ref[:, b * U:(b + 1) * U]
        c_b = jnp.tanh(z_b - aggz_b + b1)
        u_b = u_ref[:, b * U:(b + 1) * U]
        ns_ref[b] = u_b * st_ref[b] + (1.0 - u_b) * c_b


def kernel(inputs, state, weights_0, bias_0, weights_1, bias_1, lap_vals,
           edge_index):
    f32 = jnp.float32
    src = edge_index[0].astype(jnp.int32)
    dst = edge_index[1].astype(jnp.int32)
    lap = lap_vals.astype(f32)

    pad = EPAD - E
    zi = jnp.zeros((pad,), jnp.int32)
    src_p = jnp.concatenate([src, zi])
    dst_p = jnp.concatenate([dst, zi])
    lap_p = jnp.concatenate([lap, jnp.zeros((pad,), f32)])
    srccat = jnp.concatenate([2 * src_p, 2 * src_p + 1]).reshape(2 * NROW, CHK)
    srcr = src_p.reshape(NROW, CHK)
    dstr = dst_p.reshape(NROW, CHK)
    lapr = lap_p.reshape(NROW, CHK)

    xip16 = jnp.concatenate(
        [inputs.reshape(B, N).T, jnp.zeros((N, 16 - B), f32)], axis=1)
    xip128 = jnp.concatenate(
        [inputs.reshape(B, N).T, jnp.zeros((N, 128 - B), f32)], axis=1)
    state3 = state.reshape(B, N, U)
    z128 = jnp.zeros((NP, 128), f32)

    w0s = weights_0[1:]
    w0i = weights_0[0:1]
    b0r = bias_0.reshape(1, 2 * U)
    w1s = weights_1[1:]
    w1i = weights_1[0:1]
    b1r = bias_1.reshape(1, U)

    xst = pl.pallas_call(
        _prep_body,
        grid=(GRID,),
        in_specs=[pl.BlockSpec((B, RBLK, U), lambda i: (0, i, 0))],
        out_specs=pl.BlockSpec((RBLK, B * U), lambda i: (i, 0)),
        out_shape=jax.ShapeDtypeStruct((N, B * U), f32),
    )(state3)
    xst2 = xst.reshape(2 * N, 128)

    aggs = _make_agg()(srccat, dstr, lapr, xst2, z128)
    aggi = _make_xi()(srcr, dstr, lapr, xip128, z128)

    z1, uu = pl.pallas_call(
        _tc1_body,
        grid=(GRID,),
        in_specs=[
            pl.BlockSpec((RBLK, 16), lambda i: (i, 0)),
            pl.BlockSpec((RBLK, 128), lambda i: (i, 0)),
            pl.BlockSpec((RBLK, 128), lambda i: (i, 0)),
            pl.BlockSpec((B, RBLK, U), lambda i: (0, i, 0)),
            pl.BlockSpec((RBLK, 128), lambda i: (i, 0)),
            pl.BlockSpec((RBLK, 128), lambda i: (i, 0)),
            pl.BlockSpec((U, 2 * U), lambda i: (0, 0)),
            pl.BlockSpec((1, 2 * U), lambda i: (0, 0)),
            pl.BlockSpec((1, 2 * U), lambda i: (0, 0)),
            pl.BlockSpec((U, U), lambda i: (0, 0)),
            pl.BlockSpec((1, U), lambda i: (0, 0)),
        ],
        out_specs=[
            pl.BlockSpec((RBLK, B * U), lambda i: (i, 0)),
            pl.BlockSpec((RBLK, B * U), lambda i: (i, 0)),
        ],
        out_shape=[
            jax.ShapeDtypeStruct((N, B * U), f32),
            jax.ShapeDtypeStruct((N, B * U), f32),
        ],
    )(xip16, aggi[0], aggi[1], state3, aggs[0], aggs[1], w0s, w0i, b0r, w1s, w1i)

    z2 = z1.reshape(2 * N, 128)
    aggz = _make_agg()(srccat, dstr, lapr, z2, z128)

    ns = pl.pallas_call(
        _tc2_body,
        grid=(GRID,),
        in_specs=[
            pl.BlockSpec((RBLK, B * U), lambda i: (i, 0)),
            pl.BlockSpec((RBLK, 128), lambda i: (i, 0)),
            pl.BlockSpec((RBLK, 128), lambda i: (i, 0)),
            pl.BlockSpec((RBLK, B * U), lambda i: (i, 0)),
            pl.BlockSpec((B, RBLK, U), lambda i: (0, i, 0)),
            pl.BlockSpec((1, U), lambda i: (0, 0)),
        ],
        out_specs=pl.BlockSpec((B, RBLK, U), lambda i: (0, i, 0)),
        out_shape=jax.ShapeDtypeStruct((B, N, U), f32),
    )(z1, aggz[0], aggz[1], uu, state3, b1r)

    return ns.reshape(B, N * U)
